# Initial kernel scaffold; baseline (speedup 1.0000x reference)
import jax
import jax.numpy as jnp
from jax.experimental import pallas as pl


def kernel(z, pos_edge_index, neg_edge_index):
    raise NotImplementedError("write your pallas kernel here")


# speedup vs baseline: 1.1252x; 1.1252x over previous
import functools

import jax
import jax.numpy as jnp
from jax import lax
from jax.experimental import pallas as pl
from jax.experimental.pallas import tpu as pltpu
from jax.experimental.pallas import tpu_sc as plsc

EPS = 1e-15
E = 320000
D = 128
NC, NS, L = 2, 16, 16
NW = NC * NS
PER_W = 2 * E // NW
C = 80
NCHUNK = PER_W // C
NGRP = C // L


def _sc_dots(z, src_ids, dst_ids):
    mesh = plsc.VectorSubcoreMesh(core_axis_name="c", subcore_axis_name="s")

    @functools.partial(
        pl.kernel,
        mesh=mesh,
        compiler_params=pltpu.CompilerParams(needs_layout_passes=False),
        out_type=jax.ShapeDtypeStruct((2 * E,), jnp.float32),
        scratch_types=[
            pltpu.VMEM((PER_W,), jnp.int32),
            pltpu.VMEM((PER_W,), jnp.int32),
            pltpu.VMEM((2, C, D), jnp.float32),
            pltpu.VMEM((2, C, D), jnp.float32),
            pltpu.VMEM((PER_W,), jnp.float32),
            pltpu.SemaphoreType.DMA,
            pltpu.SemaphoreType.DMA,
            pltpu.SemaphoreType.DMA,
            pltpu.SemaphoreType.DMA,
        ],
    )
    def sc_kernel(z_hbm, src_hbm, dst_hbm, out_hbm,
                  sidx, didx, srows, drows, outv,
                  sem_s0, sem_d0, sem_s1, sem_d1):
        wid = lax.axis_index("s") * NC + lax.axis_index("c")
        base = wid * PER_W
        sems = ((sem_s0, sem_d0), (sem_s1, sem_d1))

        pltpu.sync_copy(src_hbm.at[pl.ds(base, PER_W)], sidx)
        pltpu.sync_copy(dst_hbm.at[pl.ds(base, PER_W)], didx)

        def fire(chunk, b):
            off = chunk * C
            pltpu.async_copy(z_hbm.at[sidx.at[pl.ds(off, C)]],
                             srows.at[b], sems[b][0])
            pltpu.async_copy(z_hbm.at[didx.at[pl.ds(off, C)]],
                             drows.at[b], sems[b][1])

        def wait(chunk, b):
            off = chunk * C
            pltpu.make_async_copy(z_hbm.at[sidx.at[pl.ds(off, C)]],
                                  srows.at[b], sems[b][0]).wait()
            pltpu.make_async_copy(z_hbm.at[didx.at[pl.ds(off, C)]],
                                  drows.at[b], sems[b][1]).wait()

        lane = lax.iota(jnp.int32, L)
        rows_g = [lane + g * L for g in range(NGRP)]

        def compute(chunk, b):
            wait(chunk, b)
            sr = srows.at[b]
            dr = drows.at[b]

            def jbody(j, accs):
                cols = jnp.full((L,), 0, jnp.int32) + j
                out = []
                for g in range(NGRP):
                    sv = plsc.load_gather(sr, [rows_g[g], cols])
                    dv = plsc.load_gather(dr, [rows_g[g], cols])
                    out.append(accs[g] + sv * dv)
                return tuple(out)

            accs = lax.fori_loop(
                0, D, jbody,
                tuple(jnp.zeros((L,), jnp.float32) for _ in range(NGRP)))
            for g in range(NGRP):
                outv[pl.ds(chunk * C + g * L, L)] = accs[g]

        fire(0, 0)

        def chunk_pair(t, carry):
            for b in range(2):
                chunk = 2 * t + b

                @pl.when(chunk + 1 < NCHUNK)
                def _():
                    fire(chunk + 1, 1 - b)

                compute(chunk, b)
            return carry

        lax.fori_loop(0, NCHUNK // 2, chunk_pair, 0)
        pltpu.sync_copy(outv, out_hbm.at[pl.ds(base, PER_W)])

    return sc_kernel(z, src_ids, dst_ids)


def _tc_loss(vals):
    v2 = vals.reshape(2 * E // D, D)
    half = E // D

    def body(v_ref, o_ref):
        v = v_ref[...]
        p = jax.nn.sigmoid(v[:half])
        q = jax.nn.sigmoid(v[half:])
        pos = -jnp.log(p + EPS)
        neg = -jnp.log(jnp.maximum(1.0 - q, 0.0))
        o_ref[0, 0] = jnp.sum(pos) / E + jnp.sum(neg) / E

    out = pl.pallas_call(
        body,
        out_shape=jax.ShapeDtypeStruct((1, 1), jnp.float32),
        out_specs=pl.BlockSpec(memory_space=pltpu.SMEM),
    )(v2)
    return out.reshape(())


def kernel(z, pos_edge_index, neg_edge_index):
    pe = pos_edge_index.astype(jnp.int32)
    ne = neg_edge_index.astype(jnp.int32)
    src = jnp.concatenate([pe[0], ne[0]])
    dst = jnp.concatenate([pe[1], ne[1]])
    dots = _sc_dots(z, src, dst)
    return _tc_loss(dots)


# speedup vs baseline: 9.4207x; 8.3723x over previous
import functools

import jax
import jax.numpy as jnp
from jax import lax
from jax.experimental import pallas as pl
from jax.experimental.pallas import tpu as pltpu
from jax.experimental.pallas import tpu_sc as plsc

EPS = 1e-15
E = 320000
D = 128
NC, NS, L = 2, 16, 16
NW = NC * NS
PER_W = 2 * E // NW
C = 80
NCHUNK = PER_W // C
NGRP = C // L


def _sc_dots(z, src_ids, dst_ids):
    mesh = plsc.VectorSubcoreMesh(core_axis_name="c", subcore_axis_name="s")

    @functools.partial(
        pl.kernel,
        mesh=mesh,
        compiler_params=pltpu.CompilerParams(needs_layout_passes=False),
        out_type=jax.ShapeDtypeStruct((2 * E,), jnp.float32),
        scratch_types=[
            pltpu.VMEM((PER_W,), jnp.int32),
            pltpu.VMEM((PER_W,), jnp.int32),
            pltpu.VMEM((2, C, D), jnp.float32),
            pltpu.VMEM((2, C, D), jnp.float32),
            pltpu.VMEM((PER_W,), jnp.float32),
            pltpu.SemaphoreType.DMA,
            pltpu.SemaphoreType.DMA,
            pltpu.SemaphoreType.DMA,
            pltpu.SemaphoreType.DMA,
        ],
    )
    def sc_kernel(z_hbm, src_hbm, dst_hbm, out_hbm,
                  sidx, didx, srows, drows, outv,
                  sem_s0, sem_d0, sem_s1, sem_d1):
        wid = lax.axis_index("s") * NC + lax.axis_index("c")
        base = wid * PER_W
        sems = ((sem_s0, sem_d0), (sem_s1, sem_d1))

        pltpu.sync_copy(src_hbm.at[pl.ds(base, PER_W)], sidx)
        pltpu.sync_copy(dst_hbm.at[pl.ds(base, PER_W)], didx)

        def fire(chunk, b):
            off = chunk * C
            pltpu.async_copy(z_hbm.at[sidx.at[pl.ds(off, C)]],
                             srows.at[b], sems[b][0])
            pltpu.async_copy(z_hbm.at[didx.at[pl.ds(off, C)]],
                             drows.at[b], sems[b][1])

        def wait(chunk, b):
            off = chunk * C
            pltpu.make_async_copy(z_hbm.at[sidx.at[pl.ds(off, C)]],
                                  srows.at[b], sems[b][0]).wait()
            pltpu.make_async_copy(z_hbm.at[didx.at[pl.ds(off, C)]],
                                  drows.at[b], sems[b][1]).wait()

        lane = lax.iota(jnp.int32, L)
        rows_g = [lane + g * L for g in range(NGRP)]

        def compute(chunk, b):
            wait(chunk, b)
            sr = srows.at[b]
            dr = drows.at[b]

            def jbody(j, accs):
                cols = (lane + j) & (D - 1)
                out = []
                for g in range(NGRP):
                    sv = plsc.load_gather(sr, [rows_g[g], cols])
                    dv = plsc.load_gather(dr, [rows_g[g], cols])
                    out.append(accs[g] + sv * dv)
                return tuple(out)

            accs = lax.fori_loop(
                0, D, jbody,
                tuple(jnp.zeros((L,), jnp.float32) for _ in range(NGRP)))
            for g in range(NGRP):
                outv[pl.ds(chunk * C + g * L, L)] = accs[g]

        fire(0, 0)

        def chunk_pair(t, carry):
            for b in range(2):
                chunk = 2 * t + b

                @pl.when(chunk + 1 < NCHUNK)
                def _():
                    fire(chunk + 1, 1 - b)

                compute(chunk, b)
            return carry

        lax.fori_loop(0, NCHUNK // 2, chunk_pair, 0)
        pltpu.sync_copy(outv, out_hbm.at[pl.ds(base, PER_W)])

    return sc_kernel(z, src_ids, dst_ids)


def _tc_loss(vals):
    v2 = vals.reshape(2 * E // D, D)
    half = E // D

    def body(v_ref, o_ref):
        v = v_ref[...]
        p = jax.nn.sigmoid(v[:half])
        q = jax.nn.sigmoid(v[half:])
        pos = -jnp.log(p + EPS)
        neg = -jnp.log(jnp.maximum(1.0 - q, 0.0))
        o_ref[0, 0] = jnp.sum(pos) / E + jnp.sum(neg) / E

    out = pl.pallas_call(
        body,
        out_shape=jax.ShapeDtypeStruct((1, 1), jnp.float32),
        out_specs=pl.BlockSpec(memory_space=pltpu.SMEM),
    )(v2)
    return out.reshape(())


def kernel(z, pos_edge_index, neg_edge_index):
    pe = pos_edge_index.astype(jnp.int32)
    ne = neg_edge_index.astype(jnp.int32)
    src = jnp.concatenate([pe[0], ne[0]])
    dst = jnp.concatenate([pe[1], ne[1]])
    dots = _sc_dots(z, src, dst)
    return _tc_loss(dots)


# speedup vs baseline: 11.4380x; 1.2141x over previous
import functools

import jax
import jax.numpy as jnp
from jax import lax
from jax.experimental import pallas as pl
from jax.experimental.pallas import tpu as pltpu
from jax.experimental.pallas import tpu_sc as plsc

EPS = 1e-15
E = 320000
D = 128
D2 = D // 2
NC, NS, L = 2, 16, 16
NW = NC * NS
PER_W = 2 * E // NW
C = 80
NCHUNK = PER_W // C
NGRP = C // L


def _sc_dots(z, src_ids, dst_ids):
    mesh = plsc.VectorSubcoreMesh(core_axis_name="c", subcore_axis_name="s")

    @functools.partial(
        pl.kernel,
        mesh=mesh,
        compiler_params=pltpu.CompilerParams(
            needs_layout_passes=False, use_tc_tiling_on_sc=False),
        out_type=jax.ShapeDtypeStruct((2 * E,), jnp.float32),
        scratch_types=[
            pltpu.VMEM((PER_W,), jnp.int32),
            pltpu.VMEM((PER_W,), jnp.int32),
            pltpu.VMEM((2, C, D2), jnp.int32),
            pltpu.VMEM((2, C, D2), jnp.int32),
            pltpu.VMEM((PER_W,), jnp.float32),
            pltpu.SemaphoreType.DMA,
            pltpu.SemaphoreType.DMA,
            pltpu.SemaphoreType.DMA,
            pltpu.SemaphoreType.DMA,
        ],
    )
    def sc_kernel(z_hbm, src_hbm, dst_hbm, out_hbm,
                  sidx, didx, srows, drows, outv,
                  sem_s0, sem_d0, sem_s1, sem_d1):
        wid = lax.axis_index("s") * NC + lax.axis_index("c")
        base = wid * PER_W
        sems = ((sem_s0, sem_d0), (sem_s1, sem_d1))

        pltpu.sync_copy(src_hbm.at[pl.ds(base, PER_W)], sidx)
        pltpu.sync_copy(dst_hbm.at[pl.ds(base, PER_W)], didx)

        def fire(chunk, b):
            off = chunk * C
            pltpu.async_copy(z_hbm.at[sidx.at[pl.ds(off, C)]],
                             srows.at[b], sems[b][0])
            pltpu.async_copy(z_hbm.at[didx.at[pl.ds(off, C)]],
                             drows.at[b], sems[b][1])

        def wait(chunk, b):
            off = chunk * C
            pltpu.make_async_copy(z_hbm.at[sidx.at[pl.ds(off, C)]],
                                  srows.at[b], sems[b][0]).wait()
            pltpu.make_async_copy(z_hbm.at[didx.at[pl.ds(off, C)]],
                                  drows.at[b], sems[b][1]).wait()

        lane = lax.iota(jnp.int32, L)
        rows_g = [lane + g * L for g in range(NGRP)]

        def compute(chunk, b):
            wait(chunk, b)
            sr = srows.at[b]
            dr = drows.at[b]

            def jbody(j, accs):
                cols = (lane + j) & (D2 - 1)
                out = []
                for g in range(NGRP):
                    ws = plsc.load_gather(sr, [rows_g[g], cols])
                    wd = plsc.load_gather(dr, [rows_g[g], cols])
                    s0, s1 = plsc.unpack(
                        plsc.bitcast(ws, jnp.bfloat16),
                        format=plsc.PackFormat.INTERLEAVED,
                        preferred_element_type=jnp.float32)
                    d0, d1 = plsc.unpack(
                        plsc.bitcast(wd, jnp.bfloat16),
                        format=plsc.PackFormat.INTERLEAVED,
                        preferred_element_type=jnp.float32)
                    out.append(accs[g] + (s0 * d0 + s1 * d1))
                return tuple(out)

            accs = lax.fori_loop(
                0, D2, jbody,
                tuple(jnp.zeros((L,), jnp.float32) for _ in range(NGRP)))
            for g in range(NGRP):
                outv[pl.ds(chunk * C + g * L, L)] = accs[g]

        fire(0, 0)

        def chunk_pair(t, carry):
            for b in range(2):
                chunk = 2 * t + b

                @pl.when(chunk + 1 < NCHUNK)
                def _():
                    fire(chunk + 1, 1 - b)

                compute(chunk, b)
            return carry

        lax.fori_loop(0, NCHUNK // 2, chunk_pair, 0)
        pltpu.sync_copy(outv, out_hbm.at[pl.ds(base, PER_W)])

    return sc_kernel(z, src_ids, dst_ids)


def _tc_loss(vals):
    v2 = vals.reshape(2 * E // D, D)
    half = E // D

    def body(v_ref, o_ref):
        v = v_ref[...]
        p = jax.nn.sigmoid(v[:half])
        q = jax.nn.sigmoid(v[half:])
        pos = -jnp.log(p + EPS)
        neg = -jnp.log(jnp.maximum(1.0 - q, 0.0))
        o_ref[0, 0] = jnp.sum(pos) / E + jnp.sum(neg) / E

    out = pl.pallas_call(
        body,
        out_shape=jax.ShapeDtypeStruct((1, 1), jnp.float32),
        out_specs=pl.BlockSpec(memory_space=pltpu.SMEM),
    )(v2)
    return out.reshape(())


def kernel(z, pos_edge_index, neg_edge_index):
    pe = pos_edge_index.astype(jnp.int32)
    ne = neg_edge_index.astype(jnp.int32)
    src = jnp.concatenate([pe[0], ne[0]])
    dst = jnp.concatenate([pe[1], ne[1]])
    zp = jax.lax.bitcast_convert_type(
        z.astype(jnp.bfloat16).reshape(z.shape[0], D2, 2), jnp.int32)
    dots = _sc_dots(zp, src, dst)
    return _tc_loss(dots)


# speedup vs baseline: 12.4398x; 1.0876x over previous
import functools

import jax
import jax.numpy as jnp
from jax import lax
from jax.experimental import pallas as pl
from jax.experimental.pallas import tpu as pltpu
from jax.experimental.pallas import tpu_sc as plsc

EPS = 1e-15
E = 320000
D = 128
D2 = D // 2
NC, NS, L = 2, 16, 16
NW = NC * NS
PER_W = 2 * E // NW
C = 128
NFULL = PER_W // C
CT = PER_W - NFULL * C


def _sc_dots(z, src_ids, dst_ids):
    mesh = plsc.VectorSubcoreMesh(core_axis_name="c", subcore_axis_name="s")

    @functools.partial(
        pl.kernel,
        mesh=mesh,
        compiler_params=pltpu.CompilerParams(
            needs_layout_passes=False, use_tc_tiling_on_sc=False),
        out_type=jax.ShapeDtypeStruct((2 * E,), jnp.float32),
        scratch_types=[
            pltpu.VMEM((PER_W,), jnp.int32),
            pltpu.VMEM((PER_W,), jnp.int32),
            pltpu.VMEM((2, C, D2), jnp.int32),
            pltpu.VMEM((2, C, D2), jnp.int32),
            pltpu.VMEM((PER_W,), jnp.float32),
            pltpu.SemaphoreType.DMA,
            pltpu.SemaphoreType.DMA,
            pltpu.SemaphoreType.DMA,
            pltpu.SemaphoreType.DMA,
        ],
    )
    def sc_kernel(z_hbm, src_hbm, dst_hbm, out_hbm,
                  sidx, didx, srows, drows, outv,
                  sem_s0, sem_d0, sem_s1, sem_d1):
        wid = lax.axis_index("s") * NC + lax.axis_index("c")
        base = wid * PER_W
        sems = ((sem_s0, sem_d0), (sem_s1, sem_d1))

        pltpu.sync_copy(src_hbm.at[pl.ds(base, PER_W)], sidx)
        pltpu.sync_copy(dst_hbm.at[pl.ds(base, PER_W)], didx)

        def fire(off, n, b):
            pltpu.async_copy(z_hbm.at[sidx.at[pl.ds(off, n)]],
                             srows.at[b, pl.ds(0, n)], sems[b][0])
            pltpu.async_copy(z_hbm.at[didx.at[pl.ds(off, n)]],
                             drows.at[b, pl.ds(0, n)], sems[b][1])

        def wait(off, n, b):
            pltpu.make_async_copy(z_hbm.at[sidx.at[pl.ds(off, n)]],
                                  srows.at[b, pl.ds(0, n)], sems[b][0]).wait()
            pltpu.make_async_copy(z_hbm.at[didx.at[pl.ds(off, n)]],
                                  drows.at[b, pl.ds(0, n)], sems[b][1]).wait()

        lane = lax.iota(jnp.int32, L)

        def compute(off, n, b):
            ngrp = n // L
            rows_g = [lane + g * L for g in range(ngrp)]
            wait(off, n, b)
            sr = srows.at[b]
            dr = drows.at[b]

            def dot_step(j, accs):
                cols = (lane + j) & (D2 - 1)
                out = []
                for g in range(ngrp):
                    ws = plsc.load_gather(sr, [rows_g[g], cols])
                    wd = plsc.load_gather(dr, [rows_g[g], cols])
                    s0, s1 = plsc.unpack(
                        plsc.bitcast(ws, jnp.bfloat16),
                        format=plsc.PackFormat.INTERLEAVED,
                        preferred_element_type=jnp.float32)
                    d0, d1 = plsc.unpack(
                        plsc.bitcast(wd, jnp.bfloat16),
                        format=plsc.PackFormat.INTERLEAVED,
                        preferred_element_type=jnp.float32)
                    out.append(accs[g] + (s0 * d0 + s1 * d1))
                return tuple(out)

            def jbody(j, accs):
                return dot_step(2 * j + 1, dot_step(2 * j, accs))

            accs = lax.fori_loop(
                0, D2 // 2, jbody,
                tuple(jnp.zeros((L,), jnp.float32) for _ in range(ngrp)))
            for g in range(ngrp):
                outv[pl.ds(off + g * L, L)] = accs[g]

        fire(0, C, 0)

        def chunk_pair(t, carry):
            for b in range(2):
                chunk = 2 * t + b

                @pl.when(chunk + 1 < NFULL)
                def _():
                    fire((chunk + 1) * C, C, 1 - b)

                compute(chunk * C, C, b)
            return carry

        lax.fori_loop(0, NFULL // 2, chunk_pair, 0)
        fire(NFULL * C, CT, 0)
        compute(NFULL * C, CT, 0)
        pltpu.sync_copy(outv, out_hbm.at[pl.ds(base, PER_W)])

    return sc_kernel(z, src_ids, dst_ids)


def _tc_loss(vals):
    v2 = vals.reshape(2 * E // D, D)
    half = E // D

    def body(v_ref, o_ref):
        v = v_ref[...]
        p = jax.nn.sigmoid(v[:half])
        q = jax.nn.sigmoid(v[half:])
        pos = -jnp.log(p + EPS)
        neg = -jnp.log(jnp.maximum(1.0 - q, 0.0))
        o_ref[0, 0] = jnp.sum(pos) / E + jnp.sum(neg) / E

    out = pl.pallas_call(
        body,
        out_shape=jax.ShapeDtypeStruct((1, 1), jnp.float32),
        out_specs=pl.BlockSpec(memory_space=pltpu.SMEM),
    )(v2)
    return out.reshape(())


def kernel(z, pos_edge_index, neg_edge_index):
    pe = pos_edge_index.astype(jnp.int32)
    ne = neg_edge_index.astype(jnp.int32)
    src = jnp.concatenate([pe[0], ne[0]])
    dst = jnp.concatenate([pe[1], ne[1]])
    zp = jax.lax.bitcast_convert_type(
        z.astype(jnp.bfloat16).reshape(z.shape[0], D2, 2), jnp.int32)
    dots = _sc_dots(zp, src, dst)
    return _tc_loss(dots)


# speedup vs baseline: 13.4513x; 1.0813x over previous
import functools

import jax
import jax.numpy as jnp
from jax import lax
from jax.experimental import pallas as pl
from jax.experimental.pallas import tpu as pltpu
from jax.experimental.pallas import tpu_sc as plsc

EPS = 1e-15
E = 320000
D = 128
D2 = D // 2
NC, NS, L = 2, 16, 16
NW = NC * NS
PER_W = 2 * E // NW
C = 128
NFULL = PER_W // C
CT = PER_W - NFULL * C


def _sc_dots(z, src_ids, dst_ids):
    mesh = plsc.VectorSubcoreMesh(core_axis_name="c", subcore_axis_name="s")

    @functools.partial(
        pl.kernel,
        mesh=mesh,
        compiler_params=pltpu.CompilerParams(
            needs_layout_passes=False, use_tc_tiling_on_sc=False),
        out_type=jax.ShapeDtypeStruct((2 * E,), jnp.float32),
        scratch_types=[
            pltpu.VMEM((PER_W,), jnp.int32),
            pltpu.VMEM((PER_W,), jnp.int32),
            pltpu.VMEM((2, C, D2), jnp.int32),
            pltpu.VMEM((2, C, D2), jnp.int32),
            pltpu.VMEM((PER_W,), jnp.float32),
            pltpu.SemaphoreType.DMA,
            pltpu.SemaphoreType.DMA,
            pltpu.SemaphoreType.DMA,
            pltpu.SemaphoreType.DMA,
        ],
    )
    def sc_kernel(z_hbm, src_hbm, dst_hbm, out_hbm,
                  sidx, didx, srows, drows, outv,
                  sem_s0, sem_d0, sem_s1, sem_d1):
        wid = lax.axis_index("s") * NC + lax.axis_index("c")
        base = wid * PER_W
        sems = ((sem_s0, sem_d0), (sem_s1, sem_d1))

        pltpu.sync_copy(src_hbm.at[pl.ds(base, PER_W)], sidx)
        pltpu.sync_copy(dst_hbm.at[pl.ds(base, PER_W)], didx)

        def fire(off, n, b):
            pltpu.async_copy(z_hbm.at[sidx.at[pl.ds(off, n)]],
                             srows.at[b, pl.ds(0, n)], sems[b][0])
            pltpu.async_copy(z_hbm.at[didx.at[pl.ds(off, n)]],
                             drows.at[b, pl.ds(0, n)], sems[b][1])

        def wait(off, n, b):
            pltpu.make_async_copy(z_hbm.at[sidx.at[pl.ds(off, n)]],
                                  srows.at[b, pl.ds(0, n)], sems[b][0]).wait()
            pltpu.make_async_copy(z_hbm.at[didx.at[pl.ds(off, n)]],
                                  drows.at[b, pl.ds(0, n)], sems[b][1]).wait()

        lane = lax.iota(jnp.int32, L)

        def compute(off, n, b):
            ngrp = n // L
            rows_g = [lane + g * L for g in range(ngrp)]
            wait(off, n, b)
            sr = srows.at[b]
            dr = drows.at[b]

            def dot_step(j, accs):
                cols = (lane + j) & (D2 - 1)
                out = []
                for g in range(ngrp):
                    ws = plsc.load_gather(sr, [rows_g[g], cols])
                    wd = plsc.load_gather(dr, [rows_g[g], cols])
                    prod = (plsc.bitcast(ws, jnp.bfloat16)
                            * plsc.bitcast(wd, jnp.bfloat16))
                    p0, p1 = plsc.unpack(
                        prod,
                        format=plsc.PackFormat.INTERLEAVED,
                        preferred_element_type=jnp.float32)
                    out.append(accs[g] + (p0 + p1))
                return tuple(out)

            def jbody(j, accs):
                return dot_step(2 * j + 1, dot_step(2 * j, accs))

            accs = lax.fori_loop(
                0, D2 // 2, jbody,
                tuple(jnp.zeros((L,), jnp.float32) for _ in range(ngrp)))
            for g in range(ngrp):
                outv[pl.ds(off + g * L, L)] = accs[g]

        fire(0, C, 0)

        def chunk_pair(t, carry):
            for b in range(2):
                chunk = 2 * t + b

                @pl.when(chunk + 1 < NFULL)
                def _():
                    fire((chunk + 1) * C, C, 1 - b)

                compute(chunk * C, C, b)
            return carry

        lax.fori_loop(0, NFULL // 2, chunk_pair, 0)
        fire(NFULL * C, CT, 0)
        compute(NFULL * C, CT, 0)
        pltpu.sync_copy(outv, out_hbm.at[pl.ds(base, PER_W)])

    return sc_kernel(z, src_ids, dst_ids)


def _tc_loss(vals):
    v2 = vals.reshape(2 * E // D, D)
    half = E // D

    def body(v_ref, o_ref):
        v = v_ref[...]
        p = jax.nn.sigmoid(v[:half])
        q = jax.nn.sigmoid(v[half:])
        pos = -jnp.log(p + EPS)
        neg = -jnp.log(jnp.maximum(1.0 - q, 0.0))
        o_ref[0, 0] = jnp.sum(pos) / E + jnp.sum(neg) / E

    out = pl.pallas_call(
        body,
        out_shape=jax.ShapeDtypeStruct((1, 1), jnp.float32),
        out_specs=pl.BlockSpec(memory_space=pltpu.SMEM),
    )(v2)
    return out.reshape(())


def kernel(z, pos_edge_index, neg_edge_index):
    pe = pos_edge_index.astype(jnp.int32)
    ne = neg_edge_index.astype(jnp.int32)
    src = jnp.concatenate([pe[0], ne[0]])
    dst = jnp.concatenate([pe[1], ne[1]])
    zp = jax.lax.bitcast_convert_type(
        z.astype(jnp.bfloat16).reshape(z.shape[0], D2, 2), jnp.int32)
    dots = _sc_dots(zp, src, dst)
    return _tc_loss(dots)


# speedup vs baseline: 14.7796x; 1.0987x over previous
import functools

import jax
import jax.numpy as jnp
from jax import lax
from jax.experimental import pallas as pl
from jax.experimental.pallas import tpu as pltpu
from jax.experimental.pallas import tpu_sc as plsc

EPS = 1e-15
E = 320000
D = 128
D2 = D // 2
NC, NS, L = 2, 16, 16
NW = NC * NS
PER_W = 2 * E // NW
C = 128
NFULL = PER_W // C
CT = PER_W - NFULL * C


def _sc_dots(z, src_ids, dst_ids):
    mesh = plsc.VectorSubcoreMesh(core_axis_name="c", subcore_axis_name="s")

    @functools.partial(
        pl.kernel,
        mesh=mesh,
        compiler_params=pltpu.CompilerParams(
            needs_layout_passes=False, use_tc_tiling_on_sc=False),
        out_type=jax.ShapeDtypeStruct((2 * E,), jnp.float32),
        scratch_types=[
            pltpu.VMEM((PER_W,), jnp.int32),
            pltpu.VMEM((PER_W,), jnp.int32),
            pltpu.VMEM((3, C, D2), jnp.int32),
            pltpu.VMEM((3, C, D2), jnp.int32),
            pltpu.VMEM((PER_W,), jnp.float32),
            pltpu.SemaphoreType.DMA,
            pltpu.SemaphoreType.DMA,
            pltpu.SemaphoreType.DMA,
            pltpu.SemaphoreType.DMA,
            pltpu.SemaphoreType.DMA,
            pltpu.SemaphoreType.DMA,
        ],
    )
    def sc_kernel(z_hbm, src_hbm, dst_hbm, out_hbm,
                  sidx, didx, srows, drows, outv,
                  sem_s0, sem_d0, sem_s1, sem_d1, sem_s2, sem_d2):
        wid = lax.axis_index("s") * NC + lax.axis_index("c")
        base = wid * PER_W
        sems = ((sem_s0, sem_d0), (sem_s1, sem_d1), (sem_s2, sem_d2))

        pltpu.sync_copy(src_hbm.at[pl.ds(base, PER_W)], sidx)
        pltpu.sync_copy(dst_hbm.at[pl.ds(base, PER_W)], didx)

        def fire(off, n, b):
            pltpu.async_copy(z_hbm.at[sidx.at[pl.ds(off, n)]],
                             srows.at[b, pl.ds(0, n)], sems[b][0])
            pltpu.async_copy(z_hbm.at[didx.at[pl.ds(off, n)]],
                             drows.at[b, pl.ds(0, n)], sems[b][1])

        def wait(off, n, b):
            pltpu.make_async_copy(z_hbm.at[sidx.at[pl.ds(off, n)]],
                                  srows.at[b, pl.ds(0, n)], sems[b][0]).wait()
            pltpu.make_async_copy(z_hbm.at[didx.at[pl.ds(off, n)]],
                                  drows.at[b, pl.ds(0, n)], sems[b][1]).wait()

        lane = lax.iota(jnp.int32, L)

        def compute(off, n, b):
            ngrp = n // L
            rows_g = [lane + g * L for g in range(ngrp)]
            wait(off, n, b)
            sr = srows.at[b]
            dr = drows.at[b]

            def dot_step(j, accs):
                cols = (lane + j) & (D2 - 1)
                out = []
                for g in range(ngrp):
                    ws = plsc.load_gather(sr, [rows_g[g], cols])
                    wd = plsc.load_gather(dr, [rows_g[g], cols])
                    prod = (plsc.bitcast(ws, jnp.bfloat16)
                            * plsc.bitcast(wd, jnp.bfloat16))
                    p0, p1 = plsc.unpack(
                        prod,
                        format=plsc.PackFormat.INTERLEAVED,
                        preferred_element_type=jnp.float32)
                    out.append(accs[g] + (p0 + p1))
                return tuple(out)

            def jbody(j, accs):
                return dot_step(2 * j + 1, dot_step(2 * j, accs))

            accs = lax.fori_loop(
                0, D2 // 2, jbody,
                tuple(jnp.zeros((L,), jnp.float32) for _ in range(ngrp)))
            for g in range(ngrp):
                outv[pl.ds(off + g * L, L)] = accs[g]

        fire(0, C, 0)
        fire(C, C, 1)

        def chunk_trip(t, carry):
            for b in range(3):
                chunk = 3 * t + b

                @pl.when(chunk + 2 < NFULL)
                def _():
                    fire((chunk + 2) * C, C, (b + 2) % 3)

                compute(chunk * C, C, b)
            return carry

        lax.fori_loop(0, NFULL // 3, chunk_trip, 0)
        fire(NFULL * C, CT, 0)
        compute(NFULL * C, CT, 0)
        pltpu.sync_copy(outv, out_hbm.at[pl.ds(base, PER_W)])

    return sc_kernel(z, src_ids, dst_ids)


def _tc_loss(vals):
    v2 = vals.reshape(2 * E // D, D)
    half = E // D

    def body(v_ref, o_ref):
        v = v_ref[...]
        p = jax.nn.sigmoid(v[:half])
        q = jax.nn.sigmoid(v[half:])
        pos = -jnp.log(p + EPS)
        neg = -jnp.log(jnp.maximum(1.0 - q, 0.0))
        o_ref[0, 0] = jnp.sum(pos) / E + jnp.sum(neg) / E

    out = pl.pallas_call(
        body,
        out_shape=jax.ShapeDtypeStruct((1, 1), jnp.float32),
        out_specs=pl.BlockSpec(memory_space=pltpu.SMEM),
    )(v2)
    return out.reshape(())


def kernel(z, pos_edge_index, neg_edge_index):
    pe = pos_edge_index.astype(jnp.int32)
    ne = neg_edge_index.astype(jnp.int32)
    src = jnp.concatenate([pe[0], ne[0]])
    dst = jnp.concatenate([pe[1], ne[1]])
    zp = jax.lax.bitcast_convert_type(
        z.astype(jnp.bfloat16).reshape(z.shape[0], D2, 2), jnp.int32)
    dots = _sc_dots(zp, src, dst)
    return _tc_loss(dots)


# speedup vs baseline: 16.1200x; 1.0907x over previous
import functools

import jax
import jax.numpy as jnp
from jax import lax
from jax.experimental import pallas as pl
from jax.experimental.pallas import tpu as pltpu
from jax.experimental.pallas import tpu_sc as plsc

EPS = 1e-15
E = 320000
D = 128
D2 = D // 2
NC, NS, L = 2, 16, 16
NW = NC * NS
PER_W = 2 * E // NW
C = 128
NFULL = PER_W // C
CT = PER_W - NFULL * C


def _sc_dots(z, pos_edges, neg_edges):
    mesh = plsc.VectorSubcoreMesh(core_axis_name="c", subcore_axis_name="s")

    @functools.partial(
        pl.kernel,
        mesh=mesh,
        compiler_params=pltpu.CompilerParams(
            needs_layout_passes=False, use_tc_tiling_on_sc=False),
        out_type=jax.ShapeDtypeStruct((2 * E,), jnp.float32),
        scratch_types=[
            pltpu.VMEM((PER_W,), jnp.int32),
            pltpu.VMEM((PER_W,), jnp.int32),
            pltpu.VMEM((3, C, D2), jnp.int32),
            pltpu.VMEM((3, C, D2), jnp.int32),
            pltpu.VMEM((PER_W,), jnp.float32),
            pltpu.SemaphoreType.DMA,
            pltpu.SemaphoreType.DMA,
            pltpu.SemaphoreType.DMA,
            pltpu.SemaphoreType.DMA,
            pltpu.SemaphoreType.DMA,
            pltpu.SemaphoreType.DMA,
        ],
    )
    def sc_kernel(z_hbm, pe_hbm, ne_hbm, out_hbm,
                  sidx, didx, srows, drows, outv,
                  sem_s0, sem_d0, sem_s1, sem_d1, sem_s2, sem_d2):
        side = lax.axis_index("c")
        lane16 = lax.axis_index("s")
        span = lane16 * PER_W
        base = side * E + span
        sems = ((sem_s0, sem_d0), (sem_s1, sem_d1), (sem_s2, sem_d2))

        @pl.when(side == 0)
        def _():
            pltpu.sync_copy(pe_hbm.at[0, pl.ds(span, PER_W)], sidx)
            pltpu.sync_copy(pe_hbm.at[1, pl.ds(span, PER_W)], didx)

        @pl.when(side == 1)
        def _():
            pltpu.sync_copy(ne_hbm.at[0, pl.ds(span, PER_W)], sidx)
            pltpu.sync_copy(ne_hbm.at[1, pl.ds(span, PER_W)], didx)

        def fire(off, n, b):
            pltpu.async_copy(z_hbm.at[sidx.at[pl.ds(off, n)]],
                             srows.at[b, pl.ds(0, n)], sems[b][0])
            pltpu.async_copy(z_hbm.at[didx.at[pl.ds(off, n)]],
                             drows.at[b, pl.ds(0, n)], sems[b][1])

        def wait(off, n, b):
            pltpu.make_async_copy(z_hbm.at[sidx.at[pl.ds(off, n)]],
                                  srows.at[b, pl.ds(0, n)], sems[b][0]).wait()
            pltpu.make_async_copy(z_hbm.at[didx.at[pl.ds(off, n)]],
                                  drows.at[b, pl.ds(0, n)], sems[b][1]).wait()

        lane = lax.iota(jnp.int32, L)

        def compute(off, n, b):
            ngrp = n // L
            rows_g = [lane + g * L for g in range(ngrp)]
            wait(off, n, b)
            sr = srows.at[b]
            dr = drows.at[b]

            def dot_step(j, accs):
                cols = (lane + j) & (D2 - 1)
                out = []
                for g in range(ngrp):
                    ws = plsc.load_gather(sr, [rows_g[g], cols])
                    wd = plsc.load_gather(dr, [rows_g[g], cols])
                    prod = (plsc.bitcast(ws, jnp.bfloat16)
                            * plsc.bitcast(wd, jnp.bfloat16))
                    p0, p1 = plsc.unpack(
                        prod,
                        format=plsc.PackFormat.INTERLEAVED,
                        preferred_element_type=jnp.float32)
                    out.append(accs[g] + (p0 + p1))
                return tuple(out)

            def jbody(j, accs):
                return dot_step(2 * j + 1, dot_step(2 * j, accs))

            accs = lax.fori_loop(
                0, D2 // 2, jbody,
                tuple(jnp.zeros((L,), jnp.float32) for _ in range(ngrp)))
            for g in range(ngrp):
                outv[pl.ds(off + g * L, L)] = accs[g]

        fire(0, C, 0)
        fire(C, C, 1)

        def chunk_trip(t, carry):
            for b in range(3):
                chunk = 3 * t + b

                @pl.when(chunk + 2 < NFULL)
                def _():
                    fire((chunk + 2) * C, C, (b + 2) % 3)

                compute(chunk * C, C, b)
            return carry

        lax.fori_loop(0, NFULL // 3, chunk_trip, 0)
        fire(NFULL * C, CT, 0)
        compute(NFULL * C, CT, 0)
        pltpu.sync_copy(outv, out_hbm.at[pl.ds(base, PER_W)])

    return sc_kernel(z, pos_edges, neg_edges)


def _tc_loss(vals):
    v2 = vals.reshape(2 * E // D, D)
    half = E // D

    def body(v_ref, o_ref):
        v = v_ref[...]
        p = jax.nn.sigmoid(v[:half])
        q = jax.nn.sigmoid(v[half:])
        pos = -jnp.log(p + EPS)
        neg = -jnp.log(jnp.maximum(1.0 - q, 0.0))
        o_ref[0, 0] = jnp.sum(pos) / E + jnp.sum(neg) / E

    out = pl.pallas_call(
        body,
        out_shape=jax.ShapeDtypeStruct((1, 1), jnp.float32),
        out_specs=pl.BlockSpec(memory_space=pltpu.SMEM),
    )(v2)
    return out.reshape(())


def kernel(z, pos_edge_index, neg_edge_index):
    pe = pos_edge_index.astype(jnp.int32)
    ne = neg_edge_index.astype(jnp.int32)
    zp = jax.lax.bitcast_convert_type(
        z.astype(jnp.bfloat16).reshape(z.shape[0], D2, 2), jnp.int32)
    dots = _sc_dots(zp, pe, ne)
    return _tc_loss(dots)


# speedup vs baseline: 18.6635x; 1.1578x over previous
import functools

import jax
import jax.numpy as jnp
from jax import lax
from jax.experimental import pallas as pl
from jax.experimental.pallas import tpu as pltpu
from jax.experimental.pallas import tpu_sc as plsc

EPS = 1e-15
E = 320000
D = 128
D2 = D // 2
NC, NS, L = 2, 16, 16
NW = NC * NS
PER_W = 2 * E // NW
C = 128
NFULL = PER_W // C
CT = PER_W - NFULL * C


def _sc_dots(z, pos_edges, neg_edges):
    mesh = plsc.VectorSubcoreMesh(core_axis_name="c", subcore_axis_name="s")

    @functools.partial(
        pl.kernel,
        mesh=mesh,
        compiler_params=pltpu.CompilerParams(
            needs_layout_passes=False, use_tc_tiling_on_sc=False),
        out_type=jax.ShapeDtypeStruct((2 * E,), jnp.float32),
        scratch_types=[
            pltpu.VMEM((PER_W,), jnp.int32),
            pltpu.VMEM((PER_W,), jnp.int32),
            pltpu.VMEM((3, C, D2), jnp.int32),
            pltpu.VMEM((3, C, D2), jnp.int32),
            pltpu.VMEM((PER_W,), jnp.float32),
            pltpu.SemaphoreType.DMA,
            pltpu.SemaphoreType.DMA,
            pltpu.SemaphoreType.DMA,
            pltpu.SemaphoreType.DMA,
            pltpu.SemaphoreType.DMA,
            pltpu.SemaphoreType.DMA,
        ],
    )
    def sc_kernel(z_hbm, pe_hbm, ne_hbm, out_hbm,
                  sidx, didx, srows, drows, outv,
                  sem_s0, sem_d0, sem_s1, sem_d1, sem_s2, sem_d2):
        z32 = z_hbm
        side = lax.axis_index("c")
        lane16 = lax.axis_index("s")
        span = lane16 * PER_W
        base = side * E + span
        sems = ((sem_s0, sem_d0), (sem_s1, sem_d1), (sem_s2, sem_d2))

        @pl.when(side == 0)
        def _():
            pltpu.sync_copy(pe_hbm.at[0, pl.ds(span, PER_W)], sidx)
            pltpu.sync_copy(pe_hbm.at[1, pl.ds(span, PER_W)], didx)

        @pl.when(side == 1)
        def _():
            pltpu.sync_copy(ne_hbm.at[0, pl.ds(span, PER_W)], sidx)
            pltpu.sync_copy(ne_hbm.at[1, pl.ds(span, PER_W)], didx)

        def fire(off, n, b):
            pltpu.async_copy(z32.at[sidx.at[pl.ds(off, n)]],
                             srows.at[b, pl.ds(0, n)], sems[b][0])
            pltpu.async_copy(z32.at[didx.at[pl.ds(off, n)]],
                             drows.at[b, pl.ds(0, n)], sems[b][1])

        def wait(off, n, b):
            pltpu.make_async_copy(z32.at[sidx.at[pl.ds(off, n)]],
                                  srows.at[b, pl.ds(0, n)], sems[b][0]).wait()
            pltpu.make_async_copy(z32.at[didx.at[pl.ds(off, n)]],
                                  drows.at[b, pl.ds(0, n)], sems[b][1]).wait()

        lane = lax.iota(jnp.int32, L)

        def compute(off, n, b):
            ngrp = n // L
            rows_g = [lane + g * L for g in range(ngrp)]
            wait(off, n, b)
            sr = srows.at[b]
            dr = drows.at[b]

            def dot_step(j, accs):
                cols = (lane + j) & (D2 - 1)
                out = []
                for g in range(ngrp):
                    ws = plsc.load_gather(sr, [rows_g[g], cols])
                    wd = plsc.load_gather(dr, [rows_g[g], cols])
                    prod = (plsc.bitcast(ws, jnp.bfloat16)
                            * plsc.bitcast(wd, jnp.bfloat16))
                    p0, p1 = plsc.unpack(
                        prod,
                        format=plsc.PackFormat.INTERLEAVED,
                        preferred_element_type=jnp.float32)
                    out.append(accs[g] + (p0 + p1))
                return tuple(out)

            def jbody(j, accs):
                return dot_step(2 * j + 1, dot_step(2 * j, accs))

            accs = lax.fori_loop(
                0, D2 // 2, jbody,
                tuple(jnp.zeros((L,), jnp.float32) for _ in range(ngrp)))
            for g in range(ngrp):
                outv[pl.ds(off + g * L, L)] = accs[g]

        fire(0, C, 0)
        fire(C, C, 1)

        def chunk_trip(t, carry):
            for b in range(3):
                chunk = 3 * t + b

                @pl.when(chunk + 2 < NFULL)
                def _():
                    fire((chunk + 2) * C, C, (b + 2) % 3)

                compute(chunk * C, C, b)
            return carry

        lax.fori_loop(0, NFULL // 3, chunk_trip, 0)
        fire(NFULL * C, CT, 0)
        compute(NFULL * C, CT, 0)
        pltpu.sync_copy(outv, out_hbm.at[pl.ds(base, PER_W)])

    return sc_kernel(z, pos_edges, neg_edges)


def _tc_pack(z):
    def body(z_ref, o_ref):
        b = jax.lax.bitcast_convert_type(
            z_ref[...].astype(jnp.bfloat16), jnp.uint16)
        lo = b[:, :D2].astype(jnp.uint32)
        hi = b[:, D2:].astype(jnp.uint32)
        o_ref[...] = jax.lax.bitcast_convert_type(
            lo | (hi << 16), jnp.int32)

    return pl.pallas_call(
        body,
        out_shape=jax.ShapeDtypeStruct((z.shape[0], D2), jnp.int32),
    )(z)


def _tc_loss(vals):
    v2 = vals.reshape(2 * E // D, D)
    half = E // D

    def body(v_ref, o_ref):
        v = v_ref[...]
        p = jax.nn.sigmoid(v[:half])
        q = jax.nn.sigmoid(v[half:])
        pos = -jnp.log(p + EPS)
        neg = -jnp.log(jnp.maximum(1.0 - q, 0.0))
        o_ref[0, 0] = jnp.sum(pos) / E + jnp.sum(neg) / E

    out = pl.pallas_call(
        body,
        out_shape=jax.ShapeDtypeStruct((1, 1), jnp.float32),
        out_specs=pl.BlockSpec(memory_space=pltpu.SMEM),
    )(v2)
    return out.reshape(())


def kernel(z, pos_edge_index, neg_edge_index):
    pe = pos_edge_index.astype(jnp.int32)
    ne = neg_edge_index.astype(jnp.int32)
    dots = _sc_dots(_tc_pack(z), pe, ne)
    return _tc_loss(dots)
